# baseline (speedup 1.0000x reference)
import functools

import jax
import jax.numpy as jnp
from jax import lax
from jax.experimental import pallas as pl
from jax.experimental.pallas import tpu as pltpu
from jax.experimental.pallas import tpu_sc as plsc

V, C, H, W = 6, 256, 58, 100
MID = 128
IMG_W, IMG_H = 1600.0, 900.0
PAD_W, PAD_H = 1600.0, 928.0

WP = 102
HP = 60
RV = HP * WP
HIN = 64
RIN = HIN * WP

NPAD = 200704
PROJ_ROWS = NPAD // 128
PROJ_BLK = 32
PROJ_GRID = PROJ_ROWS // PROJ_BLK

NW = 32
PTS_PER_W = NPAD // NW
KCH = 64
NCH = PTS_PER_W // KCH

MT = 2000
NREAL = 200000
MG = NREAL // MT


def _conv_body(x_ref, w_ref, b_ref, o_ref):
    acc = jnp.zeros((RV, MID), jnp.float32)
    k = 0
    for dy in range(3):
        for dx in range(3):
            off = dy * WP + dx
            acc = acc + jnp.dot(x_ref[0, pl.ds(off, RV), :], w_ref[k],
                                preferred_element_type=jnp.float32)
            k += 1
    o_ref[0] = acc + b_ref[0, :][None, :]


def _proj_body(m_ref, p_ref, idx_ref, w_ref):
    xx = p_ref[0].astype(jnp.bfloat16).astype(jnp.float32)
    yy = p_ref[1].astype(jnp.bfloat16).astype(jnp.float32)
    zz = p_ref[2].astype(jnp.bfloat16).astype(jnp.float32)
    shp = xx.shape
    selx = jnp.zeros(shp, jnp.float32)
    sely = jnp.zeros(shp, jnp.float32)
    fv = jnp.zeros(shp, jnp.int32)
    found = jnp.zeros(shp, jnp.bool_)
    for v in range(V):
        def m(i, j, v=v):
            return m_ref[v * 16 + i * 4 + j]
        px = m(0, 0) * xx + m(0, 1) * yy + m(0, 2) * zz + m(0, 3)
        py = m(1, 0) * xx + m(1, 1) * yy + m(1, 2) * zz + m(1, 3)
        pz = m(2, 0) * xx + m(2, 1) * yy + m(2, 2) * zz + m(2, 3)
        zc = jnp.where(pz == 0.0, 1e-9, pz)
        cx = px / zc
        cy = py / zc
        val = (cx <= IMG_W) & (cy <= IMG_H) & (cx >= 0.0) & (cy >= 0.0)
        take = val & jnp.logical_not(found)
        selx = jnp.where(take, cx, selx)
        sely = jnp.where(take, cy, sely)
        fv = jnp.where(take, v, fv)
        found = found | val
    gx = selx / PAD_W * 2.0 - 1.0
    gy = sely / PAD_H * 2.0 - 1.0
    ix = (gx + 1.0) * 0.5 * (W - 1.0)
    iy = (gy + 1.0) * 0.5 * (H - 1.0)
    x0 = jnp.floor(ix)
    y0 = jnp.floor(iy)
    x1 = x0 + 1.0
    y1 = y0 + 1.0
    wx1 = ix - x0
    wx0 = 1.0 - wx1
    wy1 = iy - y0
    wy0 = 1.0 - wy1
    pid = pl.program_id(0)
    row = lax.broadcasted_iota(jnp.int32, shp, 0)
    lane = lax.broadcasted_iota(jnp.int32, shp, 1)
    gidx = pid * (PROJ_BLK * 128) + row * 128 + lane
    livef = ((gidx < NREAL) & found).astype(jnp.float32)
    base = fv * RV
    corners = [(y0, x0, wy0 * wx0), (y0, x1, wy0 * wx1),
               (y1, x0, wy1 * wx0), (y1, x1, wy1 * wx1)]
    for c, (yf, xf, wgt) in enumerate(corners):
        inb = (xf >= 0.0) & (xf <= W - 1.0) & (yf >= 0.0) & (yf <= H - 1.0)
        xc = jnp.clip(xf, 0.0, W - 1.0).astype(jnp.int32)
        yc = jnp.clip(yf, 0.0, H - 1.0).astype(jnp.int32)
        idx_ref[c] = base + yc * WP + xc
        w_ref[c] = wgt * inb.astype(jnp.float32) * livef


def _splat(vec, l):
    return lax.gather(
        vec, jnp.full((16, 1), l, jnp.int32),
        dimension_numbers=lax.GatherDimensionNumbers(
            offset_dims=(), collapsed_slice_dims=(0,), start_index_map=(0,)),
        slice_sizes=(1,), mode=lax.GatherScatterMode.PROMISE_IN_BOUNDS)


def _sc_body(table_h, idx_h, w_h, out_h, idx_v, w_v, g_v, o_v, gsem, osem):
    wid = lax.axis_index("s") * 2 + lax.axis_index("c")
    base0 = wid * PTS_PER_W
    ibase0 = 4 * base0

    pltpu.sync_copy(idx_h.at[pl.ds(ibase0, 4 * PTS_PER_W)], idx_v)

    def fire(g, b):
        off = g * (4 * KCH)
        pltpu.async_copy(w_h.at[pl.ds(ibase0 + off, 4 * KCH)], w_v.at[b],
                         gsem.at[b])
        pltpu.async_copy(table_h.at[idx_v.at[pl.ds(off, 4 * KCH)]],
                         g_v.at[b], gsem.at[b])

    def wait_fired(g, b):
        off = g * (4 * KCH)
        pltpu.make_async_copy(w_h.at[pl.ds(ibase0 + off, 4 * KCH)],
                              w_v.at[b], gsem.at[b]).wait()
        pltpu.make_async_copy(table_h.at[idx_v.at[pl.ds(off, 4 * KCH)]],
                              g_v.at[b], gsem.at[b]).wait()

    fire(0, 0)

    def chunk(g, carry):
        b = lax.rem(g, 2)

        @pl.when(g + 1 < NCH)
        def _():
            fire(g + 1, lax.rem(g + 1, 2))

        wait_fired(g, b)

        @pl.when(g >= 2)
        def _():
            pltpu.make_async_copy(
                o_v.at[b],
                out_h.at[pl.ds(base0 + (g - 2) * KCH, KCH)],
                osem.at[b]).wait()

        for gi in range(KCH // 16):
            gbase = gi * 16
            wg = [w_v[b, pl.ds(c * KCH + gbase, 16)] for c in range(4)]

            @plsc.parallel_loop(0, 16, unroll=8)
            def _lane(l, gbase=gbase, wg=wg):
                p = gbase + l
                s0 = _splat(wg[0], l)
                s1 = _splat(wg[1], l)
                s2 = _splat(wg[2], l)
                s3 = _splat(wg[3], l)
                for r in range(8):
                    sl = pl.ds(r * 16, 16)
                    o_v[b, p, sl] = (
                        g_v[b, 0 * KCH + p, sl] * s0
                        + g_v[b, 1 * KCH + p, sl] * s1
                        + g_v[b, 2 * KCH + p, sl] * s2
                        + g_v[b, 3 * KCH + p, sl] * s3)

        pltpu.async_copy(o_v.at[b], out_h.at[pl.ds(base0 + g * KCH, KCH)],
                         osem.at[b])
        return carry

    lax.fori_loop(0, NCH, chunk, 0)
    for t in (NCH - 2, NCH - 1):
        pltpu.make_async_copy(
            o_v.at[t % 2],
            out_h.at[pl.ds(base0 + t * KCH, KCH)],
            osem.at[t % 2]).wait()


def _make_sc_gather():
    return functools.partial(
        pl.kernel,
        out_type=jax.ShapeDtypeStruct((NPAD, MID), jnp.float32),
        mesh=plsc.VectorSubcoreMesh(core_axis_name="c", subcore_axis_name="s",
                                    num_cores=2, num_subcores=16),
        scratch_types=[
            pltpu.VMEM((4 * PTS_PER_W,), jnp.int32),
            pltpu.VMEM((2, 4 * KCH), jnp.float32),
            pltpu.VMEM((2, 4 * KCH, MID), jnp.float32),
            pltpu.VMEM((2, KCH, MID), jnp.float32),
            pltpu.SemaphoreType.DMA((2,)),
            pltpu.SemaphoreType.DMA((2,)),
        ],
    )(_sc_body)


def _mom_img_body(ip_ref, wi_ref, o_ref):
    a = jnp.dot(ip_ref[...].astype(jnp.bfloat16),
                wi_ref[...].astype(jnp.bfloat16),
                preferred_element_type=jnp.float32)
    rows = jnp.concatenate([
        jnp.sum(a, axis=0)[None, :], jnp.sum(a * a, axis=0)[None, :],
        jnp.zeros((6, MID), jnp.float32)], axis=0)

    @pl.when(pl.program_id(0) == 0)
    def _():
        o_ref[...] = rows

    @pl.when(pl.program_id(0) != 0)
    def _():
        o_ref[...] = o_ref[...] + rows


def _mom_pts_body(pf_ref, wp_ref, o_ref):
    a = jnp.dot(pf_ref[...].astype(jnp.bfloat16),
                wp_ref[...].astype(jnp.bfloat16),
                preferred_element_type=jnp.float32)
    rows = jnp.concatenate([
        jnp.sum(a, axis=0)[None, :], jnp.sum(a * a, axis=0)[None, :],
        jnp.zeros((6, MID), jnp.float32)], axis=0)

    @pl.when(pl.program_id(0) == 0)
    def _():
        o_ref[...] = rows

    @pl.when(pl.program_id(0) != 0)
    def _():
        o_ref[...] = o_ref[...] + rows


def _fin_body(ip_ref, pf_ref, wi_ref, wp_ref, c_ref, o_ref):
    t1 = jnp.dot(ip_ref[...].astype(jnp.bfloat16),
                 wi_ref[...].astype(jnp.bfloat16),
                 preferred_element_type=jnp.float32)
    t2 = jnp.dot(pf_ref[...].astype(jnp.bfloat16),
                 wp_ref[...].astype(jnp.bfloat16),
                 preferred_element_type=jnp.float32)
    t = (t1 * c_ref[0, :][None, :] + t2 * c_ref[1, :][None, :]
         + c_ref[2, :][None, :])
    o_ref[...] = jnp.maximum(t, 0.0)


def kernel(img_feats, points, pts_feats, lidar2img_rts, lateral_w, lateral_b,
           img_tf_w, img_bn_gamma, img_bn_beta, pts_tf_w, pts_bn_gamma,
           pts_bn_beta):
    n = points.shape[0]

    xt = jnp.transpose(img_feats, (0, 2, 3, 1))
    xp = jnp.pad(xt, ((0, 0), (1, HIN - 1 - H), (1, WP - 1 - W), (0, 0)))
    xp = xp.reshape(V, RIN, C).astype(jnp.bfloat16)
    w9 = jnp.transpose(lateral_w, (2, 3, 1, 0)).reshape(9, C, MID) \
        .astype(jnp.bfloat16)
    table = pl.pallas_call(
        _conv_body,
        grid=(V,),
        in_specs=[pl.BlockSpec((1, RIN, C), lambda v: (v, 0, 0)),
                  pl.BlockSpec((9, C, MID), lambda v: (0, 0, 0)),
                  pl.BlockSpec((1, MID), lambda v: (0, 0))],
        out_specs=pl.BlockSpec((1, RV, MID), lambda v: (v, 0, 0)),
        out_shape=jax.ShapeDtypeStruct((V, RV, MID), jnp.float32),
    )(xp, w9, lateral_b.reshape(1, MID))
    table = table.reshape(V * RV, MID)

    p3 = jnp.pad(points, ((0, NPAD - n), (0, 0))).T.reshape(3, PROJ_ROWS, 128)
    mbits = jax.lax.bitcast_convert_type(lidar2img_rts, jnp.uint32)
    mbits = (mbits + jnp.uint32(0x7FFF) + ((mbits >> 16) & jnp.uint32(1))) \
        & jnp.uint32(0xFFFF0000)
    mflat = jax.lax.bitcast_convert_type(mbits, jnp.float32).reshape(96)
    idx4, w4 = pl.pallas_call(
        _proj_body,
        grid=(PROJ_GRID,),
        in_specs=[pl.BlockSpec(memory_space=pltpu.SMEM),
                  pl.BlockSpec((3, PROJ_BLK, 128), lambda i: (0, i, 0))],
        out_specs=[pl.BlockSpec((4, PROJ_BLK, 128), lambda i: (0, i, 0)),
                   pl.BlockSpec((4, PROJ_BLK, 128), lambda i: (0, i, 0))],
        out_shape=[jax.ShapeDtypeStruct((4, PROJ_ROWS, 128), jnp.int32),
                   jax.ShapeDtypeStruct((4, PROJ_ROWS, 128), jnp.float32)],
    )(mflat, p3)
    iflat = idx4.reshape(4, NW * NCH, KCH).transpose(1, 0, 2).reshape(-1)
    wflat = w4.reshape(4, NW * NCH, KCH).transpose(1, 0, 2).reshape(-1)

    wi = img_tf_w.T
    wp = pts_tf_w.T
    mom_pts = pl.pallas_call(
        _mom_pts_body,
        grid=(MG,),
        in_specs=[pl.BlockSpec((MT, MID), lambda i: (i, 0)),
                  pl.BlockSpec((MID, MID), lambda i: (0, 0))],
        out_specs=pl.BlockSpec((8, MID), lambda i: (0, 0)),
        out_shape=jax.ShapeDtypeStruct((8, MID), jnp.float32),
    )(pts_feats, wp)
    img_pts = _make_sc_gather()(table, iflat, wflat)

    mom_img = pl.pallas_call(
        _mom_img_body,
        grid=(MG,),
        in_specs=[pl.BlockSpec((MT, MID), lambda i: (i, 0)),
                  pl.BlockSpec((MID, MID), lambda i: (0, 0))],
        out_specs=pl.BlockSpec((8, MID), lambda i: (0, 0)),
        out_shape=jax.ShapeDtypeStruct((8, MID), jnp.float32),
    )(img_pts, wi)
    nf = jnp.float32(n)
    m1 = mom_img[0] / nf
    v1 = mom_img[1] / nf - m1 * m1
    m2 = mom_pts[0] / nf
    v2 = mom_pts[1] / nf - m2 * m2
    a1 = img_bn_gamma / jnp.sqrt(v1 + 1e-3)
    a2 = pts_bn_gamma / jnp.sqrt(v2 + 1e-3)
    cvec = (img_bn_beta - m1 * a1) + (pts_bn_beta - m2 * a2)
    coefs = jnp.concatenate([a1.reshape(1, MID), a2.reshape(1, MID),
                             cvec.reshape(1, MID),
                             jnp.zeros((5, MID), jnp.float32)], axis=0)

    out = pl.pallas_call(
        _fin_body,
        grid=(MG,),
        in_specs=[pl.BlockSpec((MT, MID), lambda i: (i, 0)),
                  pl.BlockSpec((MT, MID), lambda i: (i, 0)),
                  pl.BlockSpec((MID, MID), lambda i: (0, 0)),
                  pl.BlockSpec((MID, MID), lambda i: (0, 0)),
                  pl.BlockSpec((8, MID), lambda i: (0, 0))],
        out_specs=pl.BlockSpec((MT, MID), lambda i: (i, 0)),
        out_shape=jax.ShapeDtypeStruct((NREAL, MID), jnp.float32),
    )(img_pts, pts_feats, wi, wp, coefs)
    return out


# speedup vs baseline: 1.0041x; 1.0041x over previous
import functools

import jax
import jax.numpy as jnp
from jax import lax
from jax.experimental import pallas as pl
from jax.experimental.pallas import tpu as pltpu
from jax.experimental.pallas import tpu_sc as plsc

V, C, H, W = 6, 256, 58, 100
MID = 128
IMG_W, IMG_H = 1600.0, 900.0
PAD_W, PAD_H = 1600.0, 928.0

WP = 102
HP = 60
RV = HP * WP
HIN = 64
RIN = HIN * WP

NPAD = 200704
PROJ_ROWS = NPAD // 128
PROJ_BLK = 32
PROJ_GRID = PROJ_ROWS // PROJ_BLK

NW = 32
PTS_PER_W = NPAD // NW
KCH = 64
NCH = PTS_PER_W // KCH

MT = 2000
NREAL = 200000
MG = NREAL // MT


def _conv_body(x_ref, w_ref, b_ref, o_ref):
    acc = jnp.zeros((RV, MID), jnp.float32)
    k = 0
    for dy in range(3):
        for dx in range(3):
            off = dy * WP + dx
            acc = acc + jnp.dot(x_ref[0, pl.ds(off, RV), :], w_ref[k],
                                preferred_element_type=jnp.float32)
            k += 1
    o_ref[0] = acc + b_ref[0, :][None, :]


def _proj_body(m_ref, p_ref, idx_ref, w_ref):
    xx = p_ref[0].astype(jnp.bfloat16).astype(jnp.float32)
    yy = p_ref[1].astype(jnp.bfloat16).astype(jnp.float32)
    zz = p_ref[2].astype(jnp.bfloat16).astype(jnp.float32)
    shp = xx.shape
    selx = jnp.zeros(shp, jnp.float32)
    sely = jnp.zeros(shp, jnp.float32)
    fv = jnp.zeros(shp, jnp.int32)
    found = jnp.zeros(shp, jnp.bool_)
    for v in range(V):
        def m(i, j, v=v):
            return m_ref[v * 16 + i * 4 + j]
        px = m(0, 0) * xx + m(0, 1) * yy + m(0, 2) * zz + m(0, 3)
        py = m(1, 0) * xx + m(1, 1) * yy + m(1, 2) * zz + m(1, 3)
        pz = m(2, 0) * xx + m(2, 1) * yy + m(2, 2) * zz + m(2, 3)
        zc = jnp.where(pz == 0.0, 1e-9, pz)
        cx = px / zc
        cy = py / zc
        val = (cx <= IMG_W) & (cy <= IMG_H) & (cx >= 0.0) & (cy >= 0.0)
        take = val & jnp.logical_not(found)
        selx = jnp.where(take, cx, selx)
        sely = jnp.where(take, cy, sely)
        fv = jnp.where(take, v, fv)
        found = found | val
    gx = selx / PAD_W * 2.0 - 1.0
    gy = sely / PAD_H * 2.0 - 1.0
    ix = (gx + 1.0) * 0.5 * (W - 1.0)
    iy = (gy + 1.0) * 0.5 * (H - 1.0)
    x0 = jnp.floor(ix)
    y0 = jnp.floor(iy)
    x1 = x0 + 1.0
    y1 = y0 + 1.0
    wx1 = ix - x0
    wx0 = 1.0 - wx1
    wy1 = iy - y0
    wy0 = 1.0 - wy1
    pid = pl.program_id(0)
    row = lax.broadcasted_iota(jnp.int32, shp, 0)
    lane = lax.broadcasted_iota(jnp.int32, shp, 1)
    gidx = pid * (PROJ_BLK * 128) + row * 128 + lane
    livef = ((gidx < NREAL) & found).astype(jnp.float32)
    base = fv * RV
    corners = [(y0, x0, wy0 * wx0), (y0, x1, wy0 * wx1),
               (y1, x0, wy1 * wx0), (y1, x1, wy1 * wx1)]
    for c, (yf, xf, wgt) in enumerate(corners):
        inb = (xf >= 0.0) & (xf <= W - 1.0) & (yf >= 0.0) & (yf <= H - 1.0)
        xc = jnp.clip(xf, 0.0, W - 1.0).astype(jnp.int32)
        yc = jnp.clip(yf, 0.0, H - 1.0).astype(jnp.int32)
        idx_ref[c] = base + yc * WP + xc
        w_ref[c] = wgt * inb.astype(jnp.float32) * livef


def _splat(vec, l):
    return lax.gather(
        vec, jnp.full((16, 1), l, jnp.int32),
        dimension_numbers=lax.GatherDimensionNumbers(
            offset_dims=(), collapsed_slice_dims=(0,), start_index_map=(0,)),
        slice_sizes=(1,), mode=lax.GatherScatterMode.PROMISE_IN_BOUNDS)


def _sc_body(table_h, idx_h, w_h, out_h, idx_v, w_v, g_v, o_v, gsem, osem):
    wid = lax.axis_index("s") * 2 + lax.axis_index("c")
    base0 = wid * PTS_PER_W
    ibase0 = 4 * base0

    pltpu.sync_copy(idx_h.at[pl.ds(ibase0, 4 * PTS_PER_W)], idx_v)

    def fire(g, b):
        off = g * (4 * KCH)
        pltpu.async_copy(w_h.at[pl.ds(ibase0 + off, 4 * KCH)], w_v.at[b],
                         gsem.at[b])
        pltpu.async_copy(table_h.at[idx_v.at[pl.ds(off, 4 * KCH)]],
                         g_v.at[b], gsem.at[b])

    def wait_fired(g, b):
        off = g * (4 * KCH)
        pltpu.make_async_copy(w_h.at[pl.ds(ibase0 + off, 4 * KCH)],
                              w_v.at[b], gsem.at[b]).wait()
        pltpu.make_async_copy(table_h.at[idx_v.at[pl.ds(off, 4 * KCH)]],
                              g_v.at[b], gsem.at[b]).wait()

    fire(0, 0)

    def chunk(g, carry):
        b = lax.rem(g, 2)

        @pl.when(g + 1 < NCH)
        def _():
            fire(g + 1, lax.rem(g + 1, 2))

        wait_fired(g, b)

        @pl.when(g >= 2)
        def _():
            pltpu.make_async_copy(
                o_v.at[b],
                out_h.at[pl.ds(base0 + (g - 2) * KCH, KCH)],
                osem.at[b]).wait()

        for gi in range(KCH // 16):
            gbase = gi * 16
            wg = [w_v[b, pl.ds(c * KCH + gbase, 16)] for c in range(4)]

            @plsc.parallel_loop(0, 16, unroll=4)
            def _lane(l, gbase=gbase, wg=wg):
                p = gbase + l
                s0 = _splat(wg[0], l)
                s1 = _splat(wg[1], l)
                s2 = _splat(wg[2], l)
                s3 = _splat(wg[3], l)
                for r in range(8):
                    sl = pl.ds(r * 16, 16)
                    o_v[b, p, sl] = (
                        g_v[b, 0 * KCH + p, sl] * s0
                        + g_v[b, 1 * KCH + p, sl] * s1
                        + g_v[b, 2 * KCH + p, sl] * s2
                        + g_v[b, 3 * KCH + p, sl] * s3)

        pltpu.async_copy(o_v.at[b], out_h.at[pl.ds(base0 + g * KCH, KCH)],
                         osem.at[b])
        return carry

    lax.fori_loop(0, NCH, chunk, 0)
    for t in (NCH - 2, NCH - 1):
        pltpu.make_async_copy(
            o_v.at[t % 2],
            out_h.at[pl.ds(base0 + t * KCH, KCH)],
            osem.at[t % 2]).wait()


def _make_sc_gather():
    return functools.partial(
        pl.kernel,
        out_type=jax.ShapeDtypeStruct((NPAD, MID), jnp.float32),
        mesh=plsc.VectorSubcoreMesh(core_axis_name="c", subcore_axis_name="s",
                                    num_cores=2, num_subcores=16),
        scratch_types=[
            pltpu.VMEM((4 * PTS_PER_W,), jnp.int32),
            pltpu.VMEM((2, 4 * KCH), jnp.float32),
            pltpu.VMEM((2, 4 * KCH, MID), jnp.float32),
            pltpu.VMEM((2, KCH, MID), jnp.float32),
            pltpu.SemaphoreType.DMA((2,)),
            pltpu.SemaphoreType.DMA((2,)),
        ],
    )(_sc_body)


def _mom_img_body(ip_ref, wi_ref, o_ref):
    a = jnp.dot(ip_ref[...].astype(jnp.bfloat16),
                wi_ref[...].astype(jnp.bfloat16),
                preferred_element_type=jnp.float32)
    rows = jnp.concatenate([
        jnp.sum(a, axis=0)[None, :], jnp.sum(a * a, axis=0)[None, :],
        jnp.zeros((6, MID), jnp.float32)], axis=0)

    @pl.when(pl.program_id(0) == 0)
    def _():
        o_ref[...] = rows

    @pl.when(pl.program_id(0) != 0)
    def _():
        o_ref[...] = o_ref[...] + rows


def _mom_pts_body(pf_ref, wp_ref, o_ref):
    a = jnp.dot(pf_ref[...].astype(jnp.bfloat16),
                wp_ref[...].astype(jnp.bfloat16),
                preferred_element_type=jnp.float32)
    rows = jnp.concatenate([
        jnp.sum(a, axis=0)[None, :], jnp.sum(a * a, axis=0)[None, :],
        jnp.zeros((6, MID), jnp.float32)], axis=0)

    @pl.when(pl.program_id(0) == 0)
    def _():
        o_ref[...] = rows

    @pl.when(pl.program_id(0) != 0)
    def _():
        o_ref[...] = o_ref[...] + rows


def _fin_body(ip_ref, pf_ref, wi_ref, wp_ref, c_ref, o_ref):
    t1 = jnp.dot(ip_ref[...].astype(jnp.bfloat16),
                 wi_ref[...].astype(jnp.bfloat16),
                 preferred_element_type=jnp.float32)
    t2 = jnp.dot(pf_ref[...].astype(jnp.bfloat16),
                 wp_ref[...].astype(jnp.bfloat16),
                 preferred_element_type=jnp.float32)
    t = (t1 * c_ref[0, :][None, :] + t2 * c_ref[1, :][None, :]
         + c_ref[2, :][None, :])
    o_ref[...] = jnp.maximum(t, 0.0)


def kernel(img_feats, points, pts_feats, lidar2img_rts, lateral_w, lateral_b,
           img_tf_w, img_bn_gamma, img_bn_beta, pts_tf_w, pts_bn_gamma,
           pts_bn_beta):
    n = points.shape[0]

    xt = jnp.transpose(img_feats, (0, 2, 3, 1))
    xp = jnp.pad(xt, ((0, 0), (1, HIN - 1 - H), (1, WP - 1 - W), (0, 0)))
    xp = xp.reshape(V, RIN, C).astype(jnp.bfloat16)
    w9 = jnp.transpose(lateral_w, (2, 3, 1, 0)).reshape(9, C, MID) \
        .astype(jnp.bfloat16)
    table = pl.pallas_call(
        _conv_body,
        grid=(V,),
        in_specs=[pl.BlockSpec((1, RIN, C), lambda v: (v, 0, 0)),
                  pl.BlockSpec((9, C, MID), lambda v: (0, 0, 0)),
                  pl.BlockSpec((1, MID), lambda v: (0, 0))],
        out_specs=pl.BlockSpec((1, RV, MID), lambda v: (v, 0, 0)),
        out_shape=jax.ShapeDtypeStruct((V, RV, MID), jnp.float32),
    )(xp, w9, lateral_b.reshape(1, MID))
    table = table.reshape(V * RV, MID)

    p3 = jnp.pad(points, ((0, NPAD - n), (0, 0))).T.reshape(3, PROJ_ROWS, 128)
    mbits = jax.lax.bitcast_convert_type(lidar2img_rts, jnp.uint32)
    mbits = (mbits + jnp.uint32(0x7FFF) + ((mbits >> 16) & jnp.uint32(1))) \
        & jnp.uint32(0xFFFF0000)
    mflat = jax.lax.bitcast_convert_type(mbits, jnp.float32).reshape(96)
    idx4, w4 = pl.pallas_call(
        _proj_body,
        grid=(PROJ_GRID,),
        in_specs=[pl.BlockSpec(memory_space=pltpu.SMEM),
                  pl.BlockSpec((3, PROJ_BLK, 128), lambda i: (0, i, 0))],
        out_specs=[pl.BlockSpec((4, PROJ_BLK, 128), lambda i: (0, i, 0)),
                   pl.BlockSpec((4, PROJ_BLK, 128), lambda i: (0, i, 0))],
        out_shape=[jax.ShapeDtypeStruct((4, PROJ_ROWS, 128), jnp.int32),
                   jax.ShapeDtypeStruct((4, PROJ_ROWS, 128), jnp.float32)],
    )(mflat, p3)
    iflat = idx4.reshape(4, NW * NCH, KCH).transpose(1, 0, 2).reshape(-1)
    wflat = w4.reshape(4, NW * NCH, KCH).transpose(1, 0, 2).reshape(-1)

    wi = img_tf_w.T
    wp = pts_tf_w.T
    mom_pts = pl.pallas_call(
        _mom_pts_body,
        grid=(MG,),
        in_specs=[pl.BlockSpec((MT, MID), lambda i: (i, 0)),
                  pl.BlockSpec((MID, MID), lambda i: (0, 0))],
        out_specs=pl.BlockSpec((8, MID), lambda i: (0, 0)),
        out_shape=jax.ShapeDtypeStruct((8, MID), jnp.float32),
    )(pts_feats, wp)
    img_pts = _make_sc_gather()(table, iflat, wflat)

    mom_img = pl.pallas_call(
        _mom_img_body,
        grid=(MG,),
        in_specs=[pl.BlockSpec((MT, MID), lambda i: (i, 0)),
                  pl.BlockSpec((MID, MID), lambda i: (0, 0))],
        out_specs=pl.BlockSpec((8, MID), lambda i: (0, 0)),
        out_shape=jax.ShapeDtypeStruct((8, MID), jnp.float32),
    )(img_pts, wi)
    nf = jnp.float32(n)
    m1 = mom_img[0] / nf
    v1 = mom_img[1] / nf - m1 * m1
    m2 = mom_pts[0] / nf
    v2 = mom_pts[1] / nf - m2 * m2
    a1 = img_bn_gamma / jnp.sqrt(v1 + 1e-3)
    a2 = pts_bn_gamma / jnp.sqrt(v2 + 1e-3)
    cvec = (img_bn_beta - m1 * a1) + (pts_bn_beta - m2 * a2)
    coefs = jnp.concatenate([a1.reshape(1, MID), a2.reshape(1, MID),
                             cvec.reshape(1, MID),
                             jnp.zeros((5, MID), jnp.float32)], axis=0)

    out = pl.pallas_call(
        _fin_body,
        grid=(MG,),
        in_specs=[pl.BlockSpec((MT, MID), lambda i: (i, 0)),
                  pl.BlockSpec((MT, MID), lambda i: (i, 0)),
                  pl.BlockSpec((MID, MID), lambda i: (0, 0)),
                  pl.BlockSpec((MID, MID), lambda i: (0, 0)),
                  pl.BlockSpec((8, MID), lambda i: (0, 0))],
        out_specs=pl.BlockSpec((MT, MID), lambda i: (i, 0)),
        out_shape=jax.ShapeDtypeStruct((NREAL, MID), jnp.float32),
    )(img_pts, pts_feats, wi, wp, coefs)
    return out


# speedup vs baseline: 1.1034x; 1.0989x over previous
import functools

import jax
import jax.numpy as jnp
from jax import lax
from jax.experimental import pallas as pl
from jax.experimental.pallas import tpu as pltpu
from jax.experimental.pallas import tpu_sc as plsc

V, C, H, W = 6, 256, 58, 100
MID = 128
IMG_W, IMG_H = 1600.0, 900.0
PAD_W, PAD_H = 1600.0, 928.0

WP = 102
HP = 60
RV = HP * WP
HIN = 64
RIN = HIN * WP

NPAD = 200704
PROJ_ROWS = NPAD // 128
PROJ_BLK = 32
PROJ_GRID = PROJ_ROWS // PROJ_BLK

NW = 32
PTS_PER_W = NPAD // NW
KCH = 64
NCH = PTS_PER_W // KCH

MT = 4000
NREAL = 200000
MG = NREAL // MT


def _conv_body(x_ref, w_ref, b_ref, o_ref):
    acc = jnp.zeros((RV, MID), jnp.float32)
    k = 0
    for dy in range(3):
        for dx in range(3):
            off = dy * WP + dx
            acc = acc + jnp.dot(x_ref[0, pl.ds(off, RV), :], w_ref[k],
                                preferred_element_type=jnp.float32)
            k += 1
    o_ref[0] = acc + b_ref[0, :][None, :]


def _proj_body(m_ref, p_ref, idx_ref, w_ref):
    xx = p_ref[0].astype(jnp.bfloat16).astype(jnp.float32)
    yy = p_ref[1].astype(jnp.bfloat16).astype(jnp.float32)
    zz = p_ref[2].astype(jnp.bfloat16).astype(jnp.float32)
    shp = xx.shape
    selx = jnp.zeros(shp, jnp.float32)
    sely = jnp.zeros(shp, jnp.float32)
    fv = jnp.zeros(shp, jnp.int32)
    found = jnp.zeros(shp, jnp.bool_)
    for v in range(V):
        def m(i, j, v=v):
            return m_ref[v * 16 + i * 4 + j]
        px = m(0, 0) * xx + m(0, 1) * yy + m(0, 2) * zz + m(0, 3)
        py = m(1, 0) * xx + m(1, 1) * yy + m(1, 2) * zz + m(1, 3)
        pz = m(2, 0) * xx + m(2, 1) * yy + m(2, 2) * zz + m(2, 3)
        zc = jnp.where(pz == 0.0, 1e-9, pz)
        cx = px / zc
        cy = py / zc
        val = (cx <= IMG_W) & (cy <= IMG_H) & (cx >= 0.0) & (cy >= 0.0)
        take = val & jnp.logical_not(found)
        selx = jnp.where(take, cx, selx)
        sely = jnp.where(take, cy, sely)
        fv = jnp.where(take, v, fv)
        found = found | val
    gx = selx / PAD_W * 2.0 - 1.0
    gy = sely / PAD_H * 2.0 - 1.0
    ix = (gx + 1.0) * 0.5 * (W - 1.0)
    iy = (gy + 1.0) * 0.5 * (H - 1.0)
    x0 = jnp.floor(ix)
    y0 = jnp.floor(iy)
    x1 = x0 + 1.0
    y1 = y0 + 1.0
    wx1 = ix - x0
    wx0 = 1.0 - wx1
    wy1 = iy - y0
    wy0 = 1.0 - wy1
    pid = pl.program_id(0)
    row = lax.broadcasted_iota(jnp.int32, shp, 0)
    lane = lax.broadcasted_iota(jnp.int32, shp, 1)
    gidx = pid * (PROJ_BLK * 128) + row * 128 + lane
    livef = ((gidx < NREAL) & found).astype(jnp.float32)
    base = fv * RV
    corners = [(y0, x0, wy0 * wx0), (y0, x1, wy0 * wx1),
               (y1, x0, wy1 * wx0), (y1, x1, wy1 * wx1)]
    for c, (yf, xf, wgt) in enumerate(corners):
        inb = (xf >= 0.0) & (xf <= W - 1.0) & (yf >= 0.0) & (yf <= H - 1.0)
        xc = jnp.clip(xf, 0.0, W - 1.0).astype(jnp.int32)
        yc = jnp.clip(yf, 0.0, H - 1.0).astype(jnp.int32)
        idx_ref[c] = base + yc * WP + xc
        w_ref[c] = wgt * inb.astype(jnp.float32) * livef


def _splat(vec, l):
    return lax.gather(
        vec, jnp.full((16, 1), l, jnp.int32),
        dimension_numbers=lax.GatherDimensionNumbers(
            offset_dims=(), collapsed_slice_dims=(0,), start_index_map=(0,)),
        slice_sizes=(1,), mode=lax.GatherScatterMode.PROMISE_IN_BOUNDS)


def _sc_body(table_h, idx_h, w_h, out_h, idx_v, w_v, g_v, o_v, gsem, osem):
    wid = lax.axis_index("s") * 2 + lax.axis_index("c")
    base0 = wid * PTS_PER_W
    ibase0 = 4 * base0

    pltpu.sync_copy(idx_h.at[pl.ds(ibase0, 4 * PTS_PER_W)], idx_v)

    def fire(g, b):
        off = g * (4 * KCH)
        pltpu.async_copy(w_h.at[pl.ds(ibase0 + off, 4 * KCH)], w_v.at[b],
                         gsem.at[b])
        pltpu.async_copy(table_h.at[idx_v.at[pl.ds(off, 4 * KCH)]],
                         g_v.at[b], gsem.at[b])

    def wait_fired(g, b):
        off = g * (4 * KCH)
        pltpu.make_async_copy(w_h.at[pl.ds(ibase0 + off, 4 * KCH)],
                              w_v.at[b], gsem.at[b]).wait()
        pltpu.make_async_copy(table_h.at[idx_v.at[pl.ds(off, 4 * KCH)]],
                              g_v.at[b], gsem.at[b]).wait()

    fire(0, 0)

    def chunk(g, carry):
        b = lax.rem(g, 2)

        @pl.when(g + 1 < NCH)
        def _():
            fire(g + 1, lax.rem(g + 1, 2))

        wait_fired(g, b)

        @pl.when(g >= 2)
        def _():
            pltpu.make_async_copy(
                o_v.at[b],
                out_h.at[pl.ds(base0 + (g - 2) * KCH, KCH)],
                osem.at[b]).wait()

        for gi in range(KCH // 16):
            gbase = gi * 16
            wg = [w_v[b, pl.ds(c * KCH + gbase, 16)] for c in range(4)]

            @plsc.parallel_loop(0, 16, unroll=4)
            def _lane(l, gbase=gbase, wg=wg):
                p = gbase + l
                s0 = _splat(wg[0], l)
                s1 = _splat(wg[1], l)
                s2 = _splat(wg[2], l)
                s3 = _splat(wg[3], l)
                for r in range(8):
                    sl = pl.ds(r * 16, 16)
                    o_v[b, p, sl] = (
                        g_v[b, 0 * KCH + p, sl] * s0
                        + g_v[b, 1 * KCH + p, sl] * s1
                        + g_v[b, 2 * KCH + p, sl] * s2
                        + g_v[b, 3 * KCH + p, sl] * s3)

        pltpu.async_copy(o_v.at[b], out_h.at[pl.ds(base0 + g * KCH, KCH)],
                         osem.at[b])
        return carry

    lax.fori_loop(0, NCH, chunk, 0)
    for t in (NCH - 2, NCH - 1):
        pltpu.make_async_copy(
            o_v.at[t % 2],
            out_h.at[pl.ds(base0 + t * KCH, KCH)],
            osem.at[t % 2]).wait()


def _make_sc_gather():
    return functools.partial(
        pl.kernel,
        out_type=jax.ShapeDtypeStruct((NPAD, MID), jnp.float32),
        mesh=plsc.VectorSubcoreMesh(core_axis_name="c", subcore_axis_name="s",
                                    num_cores=2, num_subcores=16),
        scratch_types=[
            pltpu.VMEM((4 * PTS_PER_W,), jnp.int32),
            pltpu.VMEM((2, 4 * KCH), jnp.float32),
            pltpu.VMEM((2, 4 * KCH, MID), jnp.float32),
            pltpu.VMEM((2, KCH, MID), jnp.float32),
            pltpu.SemaphoreType.DMA((2,)),
            pltpu.SemaphoreType.DMA((2,)),
        ],
    )(_sc_body)


def _mom_body(ip_ref, pf_ref, wi_ref, wp_ref, o_ref):
    a = jnp.dot(ip_ref[...].astype(jnp.bfloat16),
                wi_ref[...].astype(jnp.bfloat16),
                preferred_element_type=jnp.float32)
    b = jnp.dot(pf_ref[...].astype(jnp.bfloat16),
                wp_ref[...].astype(jnp.bfloat16),
                preferred_element_type=jnp.float32)
    rows = jnp.concatenate([
        jnp.sum(a, axis=0)[None, :], jnp.sum(a * a, axis=0)[None, :],
        jnp.sum(b, axis=0)[None, :], jnp.sum(b * b, axis=0)[None, :],
        jnp.zeros((4, MID), jnp.float32)], axis=0)

    @pl.when(pl.program_id(0) == 0)
    def _():
        o_ref[...] = rows

    @pl.when(pl.program_id(0) != 0)
    def _():
        o_ref[...] = o_ref[...] + rows


def _fin_body(ip_ref, pf_ref, wi_ref, wp_ref, c_ref, o_ref):
    t1 = jnp.dot(ip_ref[...].astype(jnp.bfloat16),
                 wi_ref[...].astype(jnp.bfloat16),
                 preferred_element_type=jnp.float32)
    t2 = jnp.dot(pf_ref[...].astype(jnp.bfloat16),
                 wp_ref[...].astype(jnp.bfloat16),
                 preferred_element_type=jnp.float32)
    t = (t1 * c_ref[0, :][None, :] + t2 * c_ref[1, :][None, :]
         + c_ref[2, :][None, :])
    o_ref[...] = jnp.maximum(t, 0.0)


def kernel(img_feats, points, pts_feats, lidar2img_rts, lateral_w, lateral_b,
           img_tf_w, img_bn_gamma, img_bn_beta, pts_tf_w, pts_bn_gamma,
           pts_bn_beta):
    n = points.shape[0]

    xt = jnp.transpose(img_feats, (0, 2, 3, 1))
    xp = jnp.pad(xt, ((0, 0), (1, HIN - 1 - H), (1, WP - 1 - W), (0, 0)))
    xp = xp.reshape(V, RIN, C).astype(jnp.bfloat16)
    w9 = jnp.transpose(lateral_w, (2, 3, 1, 0)).reshape(9, C, MID) \
        .astype(jnp.bfloat16)
    table = pl.pallas_call(
        _conv_body,
        grid=(V,),
        in_specs=[pl.BlockSpec((1, RIN, C), lambda v: (v, 0, 0)),
                  pl.BlockSpec((9, C, MID), lambda v: (0, 0, 0)),
                  pl.BlockSpec((1, MID), lambda v: (0, 0))],
        out_specs=pl.BlockSpec((1, RV, MID), lambda v: (v, 0, 0)),
        out_shape=jax.ShapeDtypeStruct((V, RV, MID), jnp.float32),
    )(xp, w9, lateral_b.reshape(1, MID))
    table = table.reshape(V * RV, MID)

    p3 = jnp.pad(points, ((0, NPAD - n), (0, 0))).T.reshape(3, PROJ_ROWS, 128)
    mbits = jax.lax.bitcast_convert_type(lidar2img_rts, jnp.uint32)
    mbits = (mbits + jnp.uint32(0x7FFF) + ((mbits >> 16) & jnp.uint32(1))) \
        & jnp.uint32(0xFFFF0000)
    mflat = jax.lax.bitcast_convert_type(mbits, jnp.float32).reshape(96)
    idx4, w4 = pl.pallas_call(
        _proj_body,
        grid=(PROJ_GRID,),
        in_specs=[pl.BlockSpec(memory_space=pltpu.SMEM),
                  pl.BlockSpec((3, PROJ_BLK, 128), lambda i: (0, i, 0))],
        out_specs=[pl.BlockSpec((4, PROJ_BLK, 128), lambda i: (0, i, 0)),
                   pl.BlockSpec((4, PROJ_BLK, 128), lambda i: (0, i, 0))],
        out_shape=[jax.ShapeDtypeStruct((4, PROJ_ROWS, 128), jnp.int32),
                   jax.ShapeDtypeStruct((4, PROJ_ROWS, 128), jnp.float32)],
    )(mflat, p3)
    iflat = idx4.reshape(4, NW * NCH, KCH).transpose(1, 0, 2).reshape(-1)
    wflat = w4.reshape(4, NW * NCH, KCH).transpose(1, 0, 2).reshape(-1)

    img_pts = _make_sc_gather()(table, iflat, wflat)

    wi = img_tf_w.T
    wp = pts_tf_w.T
    mom = pl.pallas_call(
        _mom_body,
        grid=(MG,),
        in_specs=[pl.BlockSpec((MT, MID), lambda i: (i, 0)),
                  pl.BlockSpec((MT, MID), lambda i: (i, 0)),
                  pl.BlockSpec((MID, MID), lambda i: (0, 0)),
                  pl.BlockSpec((MID, MID), lambda i: (0, 0))],
        out_specs=pl.BlockSpec((8, MID), lambda i: (0, 0)),
        out_shape=jax.ShapeDtypeStruct((8, MID), jnp.float32),
    )(img_pts, pts_feats, wi, wp)
    nf = jnp.float32(n)
    m1 = mom[0] / nf
    v1 = mom[1] / nf - m1 * m1
    m2 = mom[2] / nf
    v2 = mom[3] / nf - m2 * m2
    a1 = img_bn_gamma / jnp.sqrt(v1 + 1e-3)
    a2 = pts_bn_gamma / jnp.sqrt(v2 + 1e-3)
    cvec = (img_bn_beta - m1 * a1) + (pts_bn_beta - m2 * a2)
    coefs = jnp.concatenate([a1.reshape(1, MID), a2.reshape(1, MID),
                             cvec.reshape(1, MID),
                             jnp.zeros((5, MID), jnp.float32)], axis=0)

    out = pl.pallas_call(
        _fin_body,
        grid=(MG,),
        in_specs=[pl.BlockSpec((MT, MID), lambda i: (i, 0)),
                  pl.BlockSpec((MT, MID), lambda i: (i, 0)),
                  pl.BlockSpec((MID, MID), lambda i: (0, 0)),
                  pl.BlockSpec((MID, MID), lambda i: (0, 0)),
                  pl.BlockSpec((8, MID), lambda i: (0, 0))],
        out_specs=pl.BlockSpec((MT, MID), lambda i: (i, 0)),
        out_shape=jax.ShapeDtypeStruct((NREAL, MID), jnp.float32),
    )(img_pts, pts_feats, wi, wp, coefs)
    return out


# speedup vs baseline: 1.1360x; 1.0295x over previous
import functools

import jax
import jax.numpy as jnp
from jax import lax
from jax.experimental import pallas as pl
from jax.experimental.pallas import tpu as pltpu
from jax.experimental.pallas import tpu_sc as plsc

V, C, H, W = 6, 256, 58, 100
MID = 128
IMG_W, IMG_H = 1600.0, 900.0
PAD_W, PAD_H = 1600.0, 928.0

WP = 102
HP = 60
RV = HP * WP
HIN = 64
RIN = HIN * WP

NPAD = 200704
PROJ_ROWS = NPAD // 128
PROJ_BLK = 32
PROJ_GRID = PROJ_ROWS // PROJ_BLK

NW = 32
PTS_PER_W = NPAD // NW
KCH = 64
NCH = PTS_PER_W // KCH

MT = 8000
NREAL = 200000
MG = NREAL // MT


def _conv_body(x_ref, w_ref, b_ref, o_ref):
    acc = jnp.zeros((RV, MID), jnp.float32)
    k = 0
    for dy in range(3):
        for dx in range(3):
            off = dy * WP + dx
            acc = acc + jnp.dot(x_ref[0, pl.ds(off, RV), :], w_ref[k],
                                preferred_element_type=jnp.float32)
            k += 1
    o_ref[0] = acc + b_ref[0, :][None, :]


def _proj_body(m_ref, p_ref, idx_ref, w_ref):
    xx = p_ref[0].astype(jnp.bfloat16).astype(jnp.float32)
    yy = p_ref[1].astype(jnp.bfloat16).astype(jnp.float32)
    zz = p_ref[2].astype(jnp.bfloat16).astype(jnp.float32)
    shp = xx.shape
    selx = jnp.zeros(shp, jnp.float32)
    sely = jnp.zeros(shp, jnp.float32)
    fv = jnp.zeros(shp, jnp.int32)
    found = jnp.zeros(shp, jnp.bool_)
    for v in range(V):
        def m(i, j, v=v):
            return m_ref[v * 16 + i * 4 + j]
        px = m(0, 0) * xx + m(0, 1) * yy + m(0, 2) * zz + m(0, 3)
        py = m(1, 0) * xx + m(1, 1) * yy + m(1, 2) * zz + m(1, 3)
        pz = m(2, 0) * xx + m(2, 1) * yy + m(2, 2) * zz + m(2, 3)
        zc = jnp.where(pz == 0.0, 1e-9, pz)
        cx = px / zc
        cy = py / zc
        val = (cx <= IMG_W) & (cy <= IMG_H) & (cx >= 0.0) & (cy >= 0.0)
        take = val & jnp.logical_not(found)
        selx = jnp.where(take, cx, selx)
        sely = jnp.where(take, cy, sely)
        fv = jnp.where(take, v, fv)
        found = found | val
    gx = selx / PAD_W * 2.0 - 1.0
    gy = sely / PAD_H * 2.0 - 1.0
    ix = (gx + 1.0) * 0.5 * (W - 1.0)
    iy = (gy + 1.0) * 0.5 * (H - 1.0)
    x0 = jnp.floor(ix)
    y0 = jnp.floor(iy)
    x1 = x0 + 1.0
    y1 = y0 + 1.0
    wx1 = ix - x0
    wx0 = 1.0 - wx1
    wy1 = iy - y0
    wy0 = 1.0 - wy1
    pid = pl.program_id(0)
    row = lax.broadcasted_iota(jnp.int32, shp, 0)
    lane = lax.broadcasted_iota(jnp.int32, shp, 1)
    gidx = pid * (PROJ_BLK * 128) + row * 128 + lane
    livef = ((gidx < NREAL) & found).astype(jnp.float32)
    base = fv * RV
    corners = [(y0, x0, wy0 * wx0), (y0, x1, wy0 * wx1),
               (y1, x0, wy1 * wx0), (y1, x1, wy1 * wx1)]
    for c, (yf, xf, wgt) in enumerate(corners):
        inb = (xf >= 0.0) & (xf <= W - 1.0) & (yf >= 0.0) & (yf <= H - 1.0)
        xc = jnp.clip(xf, 0.0, W - 1.0).astype(jnp.int32)
        yc = jnp.clip(yf, 0.0, H - 1.0).astype(jnp.int32)
        idx_ref[c] = base + yc * WP + xc
        w_ref[c] = wgt * inb.astype(jnp.float32) * livef


def _splat(vec, l):
    return lax.gather(
        vec, jnp.full((16, 1), l, jnp.int32),
        dimension_numbers=lax.GatherDimensionNumbers(
            offset_dims=(), collapsed_slice_dims=(0,), start_index_map=(0,)),
        slice_sizes=(1,), mode=lax.GatherScatterMode.PROMISE_IN_BOUNDS)


def _sc_body(table_h, idx_h, w_h, out_h, idx_v, w_v, g_v, o_v, gsem, osem):
    wid = lax.axis_index("s") * 2 + lax.axis_index("c")
    base0 = wid * PTS_PER_W
    ibase0 = 4 * base0

    pltpu.sync_copy(idx_h.at[pl.ds(ibase0, 4 * PTS_PER_W)], idx_v)

    def fire(g, b):
        off = g * (4 * KCH)
        pltpu.async_copy(w_h.at[pl.ds(ibase0 + off, 4 * KCH)], w_v.at[b],
                         gsem.at[b])
        pltpu.async_copy(table_h.at[idx_v.at[pl.ds(off, 4 * KCH)]],
                         g_v.at[b], gsem.at[b])

    def wait_fired(g, b):
        off = g * (4 * KCH)
        pltpu.make_async_copy(w_h.at[pl.ds(ibase0 + off, 4 * KCH)],
                              w_v.at[b], gsem.at[b]).wait()
        pltpu.make_async_copy(table_h.at[idx_v.at[pl.ds(off, 4 * KCH)]],
                              g_v.at[b], gsem.at[b]).wait()

    fire(0, 0)

    def chunk(g, carry):
        b = lax.rem(g, 2)

        @pl.when(g + 1 < NCH)
        def _():
            fire(g + 1, lax.rem(g + 1, 2))

        wait_fired(g, b)

        @pl.when(g >= 2)
        def _():
            pltpu.make_async_copy(
                o_v.at[b],
                out_h.at[pl.ds(base0 + (g - 2) * KCH, KCH)],
                osem.at[b]).wait()

        for gi in range(KCH // 16):
            gbase = gi * 16
            wg = [w_v[b, pl.ds(c * KCH + gbase, 16)] for c in range(4)]

            @plsc.parallel_loop(0, 16, unroll=4)
            def _lane(l, gbase=gbase, wg=wg):
                p = gbase + l
                s0 = _splat(wg[0], l)
                s1 = _splat(wg[1], l)
                s2 = _splat(wg[2], l)
                s3 = _splat(wg[3], l)
                for r in range(8):
                    sl = pl.ds(r * 16, 16)
                    o_v[b, p, sl] = (
                        g_v[b, 0 * KCH + p, sl] * s0
                        + g_v[b, 1 * KCH + p, sl] * s1
                        + g_v[b, 2 * KCH + p, sl] * s2
                        + g_v[b, 3 * KCH + p, sl] * s3)

        pltpu.async_copy(o_v.at[b], out_h.at[pl.ds(base0 + g * KCH, KCH)],
                         osem.at[b])
        return carry

    lax.fori_loop(0, NCH, chunk, 0)
    for t in (NCH - 2, NCH - 1):
        pltpu.make_async_copy(
            o_v.at[t % 2],
            out_h.at[pl.ds(base0 + t * KCH, KCH)],
            osem.at[t % 2]).wait()


def _make_sc_gather():
    return functools.partial(
        pl.kernel,
        out_type=jax.ShapeDtypeStruct((NPAD, MID), jnp.float32),
        mesh=plsc.VectorSubcoreMesh(core_axis_name="c", subcore_axis_name="s",
                                    num_cores=2, num_subcores=16),
        scratch_types=[
            pltpu.VMEM((4 * PTS_PER_W,), jnp.int32),
            pltpu.VMEM((2, 4 * KCH), jnp.float32),
            pltpu.VMEM((2, 4 * KCH, MID), jnp.float32),
            pltpu.VMEM((2, KCH, MID), jnp.float32),
            pltpu.SemaphoreType.DMA((2,)),
            pltpu.SemaphoreType.DMA((2,)),
        ],
    )(_sc_body)


def _mom_body(ip_ref, pf_ref, wi_ref, wp_ref, o_ref):
    a = jnp.dot(ip_ref[...].astype(jnp.bfloat16),
                wi_ref[...].astype(jnp.bfloat16),
                preferred_element_type=jnp.float32)
    b = jnp.dot(pf_ref[...].astype(jnp.bfloat16),
                wp_ref[...].astype(jnp.bfloat16),
                preferred_element_type=jnp.float32)
    rows = jnp.concatenate([
        jnp.sum(a, axis=0)[None, :], jnp.sum(a * a, axis=0)[None, :],
        jnp.sum(b, axis=0)[None, :], jnp.sum(b * b, axis=0)[None, :],
        jnp.zeros((4, MID), jnp.float32)], axis=0)

    @pl.when(pl.program_id(0) == 0)
    def _():
        o_ref[...] = rows

    @pl.when(pl.program_id(0) != 0)
    def _():
        o_ref[...] = o_ref[...] + rows


def _fin_body(ip_ref, pf_ref, wi_ref, wp_ref, c_ref, o_ref):
    t1 = jnp.dot(ip_ref[...].astype(jnp.bfloat16),
                 wi_ref[...].astype(jnp.bfloat16),
                 preferred_element_type=jnp.float32)
    t2 = jnp.dot(pf_ref[...].astype(jnp.bfloat16),
                 wp_ref[...].astype(jnp.bfloat16),
                 preferred_element_type=jnp.float32)
    t = (t1 * c_ref[0, :][None, :] + t2 * c_ref[1, :][None, :]
         + c_ref[2, :][None, :])
    o_ref[...] = jnp.maximum(t, 0.0)


def kernel(img_feats, points, pts_feats, lidar2img_rts, lateral_w, lateral_b,
           img_tf_w, img_bn_gamma, img_bn_beta, pts_tf_w, pts_bn_gamma,
           pts_bn_beta):
    n = points.shape[0]

    xt = jnp.transpose(img_feats, (0, 2, 3, 1))
    xp = jnp.pad(xt, ((0, 0), (1, HIN - 1 - H), (1, WP - 1 - W), (0, 0)))
    xp = xp.reshape(V, RIN, C).astype(jnp.bfloat16)
    w9 = jnp.transpose(lateral_w, (2, 3, 1, 0)).reshape(9, C, MID) \
        .astype(jnp.bfloat16)
    table = pl.pallas_call(
        _conv_body,
        grid=(V,),
        in_specs=[pl.BlockSpec((1, RIN, C), lambda v: (v, 0, 0)),
                  pl.BlockSpec((9, C, MID), lambda v: (0, 0, 0)),
                  pl.BlockSpec((1, MID), lambda v: (0, 0))],
        out_specs=pl.BlockSpec((1, RV, MID), lambda v: (v, 0, 0)),
        out_shape=jax.ShapeDtypeStruct((V, RV, MID), jnp.float32),
    )(xp, w9, lateral_b.reshape(1, MID))
    table = table.reshape(V * RV, MID)

    p3 = jnp.pad(points, ((0, NPAD - n), (0, 0))).T.reshape(3, PROJ_ROWS, 128)
    mbits = jax.lax.bitcast_convert_type(lidar2img_rts, jnp.uint32)
    mbits = (mbits + jnp.uint32(0x7FFF) + ((mbits >> 16) & jnp.uint32(1))) \
        & jnp.uint32(0xFFFF0000)
    mflat = jax.lax.bitcast_convert_type(mbits, jnp.float32).reshape(96)
    idx4, w4 = pl.pallas_call(
        _proj_body,
        grid=(PROJ_GRID,),
        in_specs=[pl.BlockSpec(memory_space=pltpu.SMEM),
                  pl.BlockSpec((3, PROJ_BLK, 128), lambda i: (0, i, 0))],
        out_specs=[pl.BlockSpec((4, PROJ_BLK, 128), lambda i: (0, i, 0)),
                   pl.BlockSpec((4, PROJ_BLK, 128), lambda i: (0, i, 0))],
        out_shape=[jax.ShapeDtypeStruct((4, PROJ_ROWS, 128), jnp.int32),
                   jax.ShapeDtypeStruct((4, PROJ_ROWS, 128), jnp.float32)],
    )(mflat, p3)
    iflat = idx4.reshape(4, NW * NCH, KCH).transpose(1, 0, 2).reshape(-1)
    wflat = w4.reshape(4, NW * NCH, KCH).transpose(1, 0, 2).reshape(-1)

    img_pts = _make_sc_gather()(table, iflat, wflat)

    wi = img_tf_w.T
    wp = pts_tf_w.T
    mom = pl.pallas_call(
        _mom_body,
        grid=(MG,),
        in_specs=[pl.BlockSpec((MT, MID), lambda i: (i, 0)),
                  pl.BlockSpec((MT, MID), lambda i: (i, 0)),
                  pl.BlockSpec((MID, MID), lambda i: (0, 0)),
                  pl.BlockSpec((MID, MID), lambda i: (0, 0))],
        out_specs=pl.BlockSpec((8, MID), lambda i: (0, 0)),
        out_shape=jax.ShapeDtypeStruct((8, MID), jnp.float32),
    )(img_pts, pts_feats, wi, wp)
    nf = jnp.float32(n)
    m1 = mom[0] / nf
    v1 = mom[1] / nf - m1 * m1
    m2 = mom[2] / nf
    v2 = mom[3] / nf - m2 * m2
    a1 = img_bn_gamma / jnp.sqrt(v1 + 1e-3)
    a2 = pts_bn_gamma / jnp.sqrt(v2 + 1e-3)
    cvec = (img_bn_beta - m1 * a1) + (pts_bn_beta - m2 * a2)
    coefs = jnp.concatenate([a1.reshape(1, MID), a2.reshape(1, MID),
                             cvec.reshape(1, MID),
                             jnp.zeros((5, MID), jnp.float32)], axis=0)

    out = pl.pallas_call(
        _fin_body,
        grid=(MG,),
        in_specs=[pl.BlockSpec((MT, MID), lambda i: (i, 0)),
                  pl.BlockSpec((MT, MID), lambda i: (i, 0)),
                  pl.BlockSpec((MID, MID), lambda i: (0, 0)),
                  pl.BlockSpec((MID, MID), lambda i: (0, 0)),
                  pl.BlockSpec((8, MID), lambda i: (0, 0))],
        out_specs=pl.BlockSpec((MT, MID), lambda i: (i, 0)),
        out_shape=jax.ShapeDtypeStruct((NREAL, MID), jnp.float32),
    )(img_pts, pts_feats, wi, wp, coefs)
    return out
